# baseline (speedup 1.0000x reference)
import functools

import jax
import jax.numpy as jnp
from jax import lax
from jax.experimental import pallas as pl
from jax.experimental.pallas import tpu as pltpu
from jax.experimental.pallas import tpu_sc as plsc

NUM_ENTITIES = 1000000
NUM_RELATIONS = 1000
D = 64
BATCH = 16384

NC, NS, L = 2, 16, 16
NW = NC * NS
CHUNK = 128
N_CHUNKS = BATCH // CHUNK
CPW = N_CHUNKS // NW


def _trig_body(rel_ref, cs_ref):
    th = rel_ref[...]
    cs_ref[...] = jnp.concatenate([jnp.cos(th), jnp.sin(th)], axis=1)


_trig = pl.pallas_call(
    _trig_body,
    out_shape=jax.ShapeDtypeStruct((NUM_RELATIONS, 2 * D), jnp.float32),
)


def _rotate_body(e1_ref, r_ref, ent_pk, cs_t,
                 out, idx_e, idx_r, pk, cs, sem):
    wid = lax.axis_index("s") * NC + lax.axis_index("c")
    row0 = wid * CPW
    pltpu.sync_copy(e1_ref.at[pl.ds(row0, CPW)], idx_e)
    pltpu.sync_copy(r_ref.at[pl.ds(row0, CPW)], idx_r)
    hi_mask = jnp.full((L,), -65536, jnp.int32)
    for j in range(CPW):
        cps = [
            pltpu.async_copy(ent_pk.at[idx_e.at[j]], pk, sem),
            pltpu.async_copy(cs_t.at[idx_r.at[j]], cs, sem),
        ]
        for c in cps:
            c.wait()

        def body(i, carry):
            for k in range(D // L):
                sl = pl.ds(k * L, L)
                sh = pl.ds(D + k * L, L)
                v = pk[i, sl]
                a = plsc.bitcast(v << 16, jnp.float32)
                b = plsc.bitcast(v & hi_mask, jnp.float32)
                c = cs[i, sl]
                s = cs[i, sh]
                cs[i, sl] = a * c - b * s
                cs[i, sh] = a * s + b * c
            return carry

        lax.fori_loop(0, CHUNK, body, 0)
        base = (row0 + j) * CHUNK
        pltpu.sync_copy(cs, out.at[pl.ds(base, CHUNK)])


_rotate = functools.partial(
    pl.kernel,
    out_type=jax.ShapeDtypeStruct((BATCH, 2 * D), jnp.float32),
    mesh=plsc.VectorSubcoreMesh(
        core_axis_name="c", subcore_axis_name="s", num_cores=NC, num_subcores=NS),
    scratch_types=[
        pltpu.VMEM((CPW, CHUNK), jnp.int32),
        pltpu.VMEM((CPW, CHUNK), jnp.int32),
        pltpu.VMEM((CHUNK, D), jnp.int32),
        pltpu.VMEM((CHUNK, 2 * D), jnp.float32),
        pltpu.SemaphoreType.DMA,
    ],
    compiler_params=pltpu.CompilerParams(
        needs_layout_passes=False, use_tc_tiling_on_sc=False),
)(_rotate_body)


def kernel(e1, r, entity_real, entity_img, relation):
    e1 = e1.astype(jnp.int32).reshape(N_CHUNKS, CHUNK)
    r = r.astype(jnp.int32).reshape(N_CHUNKS, CHUNK)
    real16 = lax.bitcast_convert_type(
        entity_real.astype(jnp.bfloat16), jnp.uint16).astype(jnp.uint32)
    img16 = lax.bitcast_convert_type(
        entity_img.astype(jnp.bfloat16), jnp.uint16).astype(jnp.uint32)
    packed = lax.bitcast_convert_type((img16 << 16) | real16, jnp.int32)
    cs_t = _trig(relation)
    out = _rotate(e1, r, packed, cs_t)
    return out[:, :D], out[:, D:]


# speedup vs baseline: 2.6282x; 2.6282x over previous
import functools

import jax
import jax.numpy as jnp
from jax import lax
from jax.experimental import pallas as pl
from jax.experimental.pallas import tpu as pltpu
from jax.experimental.pallas import tpu_sc as plsc

NUM_ENTITIES = 1000000
NUM_RELATIONS = 1000
D = 64
BATCH = 16384
HALF = NUM_ENTITIES // 2

NC, NS, L = 2, 16, 16
NW = NC * NS
CHUNK = 128
N_CHUNKS = BATCH // CHUNK
CPW = N_CHUNKS // NW

GRP = 4096
PAIR = GRP // 2
PSTEPS = -(-NUM_ENTITIES // GRP)


def _trig_body(rel_ref, cs_ref):
    th = rel_ref[...]
    cs_ref[...] = jnp.concatenate([jnp.cos(th), jnp.sin(th)], axis=1)


_trig = pl.pallas_call(
    _trig_body,
    out_shape=jax.ShapeDtypeStruct((NUM_RELATIONS, 2 * D), jnp.float32),
)


def _pack_pair(re, im):
    reu = lax.bitcast_convert_type(
        re.astype(jnp.bfloat16), jnp.uint16).astype(jnp.uint32)
    imu = lax.bitcast_convert_type(
        im.astype(jnp.bfloat16), jnp.uint16).astype(jnp.uint32)
    return (imu << 16) | reu


def _pack_body(re_ref, im_ref, out_ref):
    t = _pack_pair(re_ref[...], im_ref[...]).T
    out_ref[...] = lax.bitcast_convert_type(
        jnp.concatenate([t[:PAIR], t[PAIR:]], axis=1), jnp.int32)


_pack = pl.pallas_call(
    _pack_body,
    grid=(PSTEPS,),
    in_specs=[
        pl.BlockSpec((D, GRP), lambda k: (0, k)),
        pl.BlockSpec((D, GRP), lambda k: (0, k)),
    ],
    out_specs=pl.BlockSpec((PAIR, 2 * D), lambda k: (k, 0)),
    out_shape=jax.ShapeDtypeStruct((HALF, 2 * D), jnp.int32),
)


def _rotate_body(sup_ref, r_ref, col_ref, ent_pk, cs_t,
                 out, idx_e, idx_r, col_v, pk, cs, sem):
    wid = lax.axis_index("s") * NC + lax.axis_index("c")
    row0 = wid * CPW
    pltpu.sync_copy(sup_ref.at[pl.ds(row0, CPW)], idx_e)
    pltpu.sync_copy(r_ref.at[pl.ds(row0, CPW)], idx_r)
    pltpu.sync_copy(col_ref.at[pl.ds(row0, CPW)], col_v)
    hi_mask = jnp.full((L,), -65536, jnp.int32)
    for j in range(CPW):
        cps = [
            pltpu.async_copy(ent_pk.at[idx_e.at[j]], pk, sem),
            pltpu.async_copy(cs_t.at[idx_r.at[j]], cs, sem),
        ]
        for c in cps:
            c.wait()

        def body(i, carry):
            zl = jnp.zeros((L,), jnp.int32)
            p = plsc.load_gather(col_v, [zl + j, zl + i])
            m = p > 0
            for k in range(D // L):
                sl = pl.ds(k * L, L)
                sh = pl.ds(D + k * L, L)
                v = jnp.where(m, pk[i, sh], pk[i, sl])
                a = plsc.bitcast(v << 16, jnp.float32)
                b = plsc.bitcast(v & hi_mask, jnp.float32)
                c = cs[i, sl]
                s = cs[i, sh]
                cs[i, sl] = a * c - b * s
                cs[i, sh] = a * s + b * c
            return carry

        lax.fori_loop(0, CHUNK, body, 0)
        base = (row0 + j) * CHUNK
        pltpu.sync_copy(cs, out.at[pl.ds(base, CHUNK)])


_rotate = functools.partial(
    pl.kernel,
    out_type=jax.ShapeDtypeStruct((BATCH, 2 * D), jnp.float32),
    mesh=plsc.VectorSubcoreMesh(
        core_axis_name="c", subcore_axis_name="s", num_cores=NC, num_subcores=NS),
    scratch_types=[
        pltpu.VMEM((CPW, CHUNK), jnp.int32),
        pltpu.VMEM((CPW, CHUNK), jnp.int32),
        pltpu.VMEM((CPW, CHUNK), jnp.int32),
        pltpu.VMEM((CHUNK, 2 * D), jnp.int32),
        pltpu.VMEM((CHUNK, 2 * D), jnp.float32),
        pltpu.SemaphoreType.DMA,
    ],
    compiler_params=pltpu.CompilerParams(needs_layout_passes=False),
)(_rotate_body)


def kernel(e1, r, entity_real, entity_img, relation):
    e1 = e1.astype(jnp.int32)
    r = r.astype(jnp.int32).reshape(N_CHUNKS, CHUNK)
    off = e1 % GRP
    sup = (PAIR * (e1 // GRP) + (off % PAIR)).reshape(N_CHUNKS, CHUNK)
    col = jnp.where(off >= PAIR, D, 0).astype(jnp.int32).reshape(N_CHUNKS, CHUNK)
    packed = _pack(entity_real.T, entity_img.T)
    cs_t = _trig(relation)
    out = _rotate(sup, r, col, packed, cs_t)
    return out[:, :D], out[:, D:]


# speedup vs baseline: 2.6319x; 1.0014x over previous
import functools

import jax
import jax.numpy as jnp
from jax import lax
from jax.experimental import pallas as pl
from jax.experimental.pallas import tpu as pltpu
from jax.experimental.pallas import tpu_sc as plsc

NUM_ENTITIES = 1000000
NUM_RELATIONS = 1000
D = 64
BATCH = 16384
HALF = NUM_ENTITIES // 2

NC, NS, L = 2, 16, 16
NW = NC * NS
CHUNK = 128
N_CHUNKS = BATCH // CHUNK
CPW = N_CHUNKS // NW

GRP = 4096
PAIR = GRP // 2
PSTEPS = -(-NUM_ENTITIES // GRP)
PROWS = PSTEPS * PAIR


def _trig_body(rel_ref, cs_ref):
    th = rel_ref[...]
    cs_ref[...] = jnp.concatenate([jnp.cos(th), jnp.sin(th)], axis=1)


_trig = pl.pallas_call(
    _trig_body,
    out_shape=jax.ShapeDtypeStruct((NUM_RELATIONS, 2 * D), jnp.float32),
)


def _pack_pair(re, im):
    reu = lax.bitcast_convert_type(
        re.astype(jnp.bfloat16), jnp.uint16).astype(jnp.uint32)
    imu = lax.bitcast_convert_type(
        im.astype(jnp.bfloat16), jnp.uint16).astype(jnp.uint32)
    return (imu << 16) | reu


def _pack_body(re_ref, im_ref, out_ref):
    t = _pack_pair(re_ref[...], im_ref[...]).T
    out_ref[...] = lax.bitcast_convert_type(
        jnp.concatenate([t[:PAIR], t[PAIR:]], axis=1), jnp.int32)


_pack = pl.pallas_call(
    _pack_body,
    grid=(PSTEPS,),
    in_specs=[
        pl.BlockSpec((D, GRP), lambda k: (0, k)),
        pl.BlockSpec((D, GRP), lambda k: (0, k)),
    ],
    out_specs=pl.BlockSpec((PAIR, 2 * D), lambda k: (k, 0)),
    out_shape=jax.ShapeDtypeStruct((PROWS, 2 * D), jnp.int32),
)


def _rotate_body(sup_ref, r_ref, col_ref, ent_pk, cs_t,
                 out, idx_e, idx_r, col_v, pk, cs, sem):
    wid = lax.axis_index("s") * NC + lax.axis_index("c")
    row0 = wid * CPW
    pltpu.sync_copy(sup_ref.at[pl.ds(row0, CPW)], idx_e)
    pltpu.sync_copy(r_ref.at[pl.ds(row0, CPW)], idx_r)
    pltpu.sync_copy(col_ref.at[pl.ds(row0, CPW)], col_v)
    hi_mask = jnp.full((L,), -65536, jnp.int32)
    for j in range(CPW):
        cps = [
            pltpu.async_copy(ent_pk.at[idx_e.at[j]], pk, sem),
            pltpu.async_copy(cs_t.at[idx_r.at[j]], cs, sem),
        ]
        for c in cps:
            c.wait()

        def body(i, carry):
            zl = jnp.zeros((L,), jnp.int32)
            p = plsc.load_gather(col_v, [zl + j, zl + i])
            m = p > 0
            for k in range(D // L):
                sl = pl.ds(k * L, L)
                sh = pl.ds(D + k * L, L)
                v = jnp.where(m, pk[i, sh], pk[i, sl])
                a = plsc.bitcast(v << 16, jnp.float32)
                b = plsc.bitcast(v & hi_mask, jnp.float32)
                c = cs[i, sl]
                s = cs[i, sh]
                cs[i, sl] = a * c - b * s
                cs[i, sh] = a * s + b * c
            return carry

        lax.fori_loop(0, CHUNK, body, 0)
        base = (row0 + j) * CHUNK
        pltpu.sync_copy(cs, out.at[pl.ds(base, CHUNK)])


_rotate = functools.partial(
    pl.kernel,
    out_type=jax.ShapeDtypeStruct((BATCH, 2 * D), jnp.float32),
    mesh=plsc.VectorSubcoreMesh(
        core_axis_name="c", subcore_axis_name="s", num_cores=NC, num_subcores=NS),
    scratch_types=[
        pltpu.VMEM((CPW, CHUNK), jnp.int32),
        pltpu.VMEM((CPW, CHUNK), jnp.int32),
        pltpu.VMEM((CPW, CHUNK), jnp.int32),
        pltpu.VMEM((CHUNK, 2 * D), jnp.int32),
        pltpu.VMEM((CHUNK, 2 * D), jnp.float32),
        pltpu.SemaphoreType.DMA,
    ],
    compiler_params=pltpu.CompilerParams(needs_layout_passes=False),
)(_rotate_body)


def kernel(e1, r, entity_real, entity_img, relation):
    e1 = e1.astype(jnp.int32)
    r = r.astype(jnp.int32).reshape(N_CHUNKS, CHUNK)
    off = e1 % GRP
    sup = (PAIR * (e1 // GRP) + (off % PAIR)).reshape(N_CHUNKS, CHUNK)
    col = jnp.where(off >= PAIR, D, 0).astype(jnp.int32).reshape(N_CHUNKS, CHUNK)
    packed = _pack(entity_real.T, entity_img.T)
    cs_t = _trig(relation)
    out = _rotate(sup, r, col, packed, cs_t)
    return out[:, :D], out[:, D:]


# speedup vs baseline: 3.1427x; 1.1941x over previous
import functools

import jax
import jax.numpy as jnp
from jax import lax
from jax.experimental import pallas as pl
from jax.experimental.pallas import tpu as pltpu
from jax.experimental.pallas import tpu_sc as plsc

NUM_ENTITIES = 1000000
NUM_RELATIONS = 1000
D = 64
BATCH = 16384
HALF = NUM_ENTITIES // 2

NC, NS, L = 2, 16, 16
NW = NC * NS
CHUNK = 128
N_CHUNKS = BATCH // CHUNK
CPW = N_CHUNKS // NW

GRP = 8192
PAIR = GRP // 2
PSTEPS = -(-NUM_ENTITIES // GRP)
PROWS = PSTEPS * PAIR


def _trig_body(rel_ref, cs_ref):
    th = rel_ref[...]
    cs_ref[...] = jnp.concatenate([jnp.cos(th), jnp.sin(th)], axis=1)


_trig = pl.pallas_call(
    _trig_body,
    out_shape=jax.ShapeDtypeStruct((NUM_RELATIONS, 2 * D), jnp.float32),
)


def _pack_pair(re, im):
    reu = lax.bitcast_convert_type(
        re.astype(jnp.bfloat16), jnp.uint16).astype(jnp.uint32)
    imu = lax.bitcast_convert_type(
        im.astype(jnp.bfloat16), jnp.uint16).astype(jnp.uint32)
    return (imu << 16) | reu


def _pack_body(re_ref, im_ref, out_ref):
    t = _pack_pair(re_ref[...], im_ref[...]).T
    out_ref[...] = lax.bitcast_convert_type(
        jnp.concatenate([t[:PAIR], t[PAIR:]], axis=1), jnp.int32)


_pack = pl.pallas_call(
    _pack_body,
    grid=(PSTEPS,),
    in_specs=[
        pl.BlockSpec((D, GRP), lambda k: (0, k)),
        pl.BlockSpec((D, GRP), lambda k: (0, k)),
    ],
    out_specs=pl.BlockSpec((PAIR, 2 * D), lambda k: (k, 0)),
    out_shape=jax.ShapeDtypeStruct((PROWS, 2 * D), jnp.int32),
)


def _rotate_body(sup_ref, r_ref, col_ref, ent_pk, cs_t,
                 out, idx_e, idx_r, col_v, pk, cs, sem):
    wid = lax.axis_index("s") * NC + lax.axis_index("c")
    row0 = wid * CPW
    pltpu.sync_copy(sup_ref.at[pl.ds(row0, CPW)], idx_e)
    pltpu.sync_copy(r_ref.at[pl.ds(row0, CPW)], idx_r)
    pltpu.sync_copy(col_ref.at[pl.ds(row0, CPW)], col_v)
    hi_mask = jnp.full((L,), -65536, jnp.int32)
    for j in range(CPW):
        cps = [
            pltpu.async_copy(ent_pk.at[idx_e.at[j]], pk, sem),
            pltpu.async_copy(cs_t.at[idx_r.at[j]], cs, sem),
        ]
        for c in cps:
            c.wait()

        def body(i, carry):
            zl = jnp.zeros((L,), jnp.int32)
            p = plsc.load_gather(col_v, [zl + j, zl + i])
            m = p > 0
            for k in range(D // L):
                sl = pl.ds(k * L, L)
                sh = pl.ds(D + k * L, L)
                v = jnp.where(m, pk[i, sh], pk[i, sl])
                a = plsc.bitcast(v << 16, jnp.float32)
                b = plsc.bitcast(v & hi_mask, jnp.float32)
                c = cs[i, sl]
                s = cs[i, sh]
                cs[i, sl] = a * c - b * s
                cs[i, sh] = a * s + b * c
            return carry

        lax.fori_loop(0, CHUNK, body, 0)
        base = (row0 + j) * CHUNK
        pltpu.sync_copy(cs, out.at[pl.ds(base, CHUNK)])


_rotate = functools.partial(
    pl.kernel,
    out_type=jax.ShapeDtypeStruct((BATCH, 2 * D), jnp.float32),
    mesh=plsc.VectorSubcoreMesh(
        core_axis_name="c", subcore_axis_name="s", num_cores=NC, num_subcores=NS),
    scratch_types=[
        pltpu.VMEM((CPW, CHUNK), jnp.int32),
        pltpu.VMEM((CPW, CHUNK), jnp.int32),
        pltpu.VMEM((CPW, CHUNK), jnp.int32),
        pltpu.VMEM((CHUNK, 2 * D), jnp.int32),
        pltpu.VMEM((CHUNK, 2 * D), jnp.float32),
        pltpu.SemaphoreType.DMA,
    ],
    compiler_params=pltpu.CompilerParams(needs_layout_passes=False),
)(_rotate_body)


def kernel(e1, r, entity_real, entity_img, relation):
    e1 = e1.astype(jnp.int32)
    r = r.astype(jnp.int32).reshape(N_CHUNKS, CHUNK)
    off = e1 % GRP
    sup = (PAIR * (e1 // GRP) + (off % PAIR)).reshape(N_CHUNKS, CHUNK)
    col = jnp.where(off >= PAIR, D, 0).astype(jnp.int32).reshape(N_CHUNKS, CHUNK)
    packed = _pack(entity_real.T, entity_img.T)
    cs_t = _trig(relation)
    out = _rotate(sup, r, col, packed, cs_t)
    return out[:, :D], out[:, D:]


# speedup vs baseline: 3.4704x; 1.1043x over previous
import functools

import jax
import jax.numpy as jnp
from jax import lax
from jax.experimental import pallas as pl
from jax.experimental.pallas import tpu as pltpu
from jax.experimental.pallas import tpu_sc as plsc

NUM_ENTITIES = 1000000
NUM_RELATIONS = 1000
D = 64
BATCH = 16384
HALF = NUM_ENTITIES // 2

NC, NS, L = 2, 16, 16
NW = NC * NS
CHUNK = 128
N_CHUNKS = BATCH // CHUNK
CPW = N_CHUNKS // NW

GRP = 16384
PAIR = GRP // 2
PSTEPS = -(-NUM_ENTITIES // GRP)
PROWS = PSTEPS * PAIR


def _trig_body(rel_ref, cs_ref):
    th = rel_ref[...]
    cs_ref[...] = jnp.concatenate([jnp.cos(th), jnp.sin(th)], axis=1)


_trig = pl.pallas_call(
    _trig_body,
    out_shape=jax.ShapeDtypeStruct((NUM_RELATIONS, 2 * D), jnp.float32),
)


def _pack_pair(re, im):
    reu = lax.bitcast_convert_type(
        re.astype(jnp.bfloat16), jnp.uint16).astype(jnp.uint32)
    imu = lax.bitcast_convert_type(
        im.astype(jnp.bfloat16), jnp.uint16).astype(jnp.uint32)
    return (imu << 16) | reu


def _pack_body(re_ref, im_ref, out_ref):
    t = _pack_pair(re_ref[...], im_ref[...]).T
    out_ref[...] = lax.bitcast_convert_type(
        jnp.concatenate([t[:PAIR], t[PAIR:]], axis=1), jnp.int32)


_pack = pl.pallas_call(
    _pack_body,
    grid=(PSTEPS,),
    in_specs=[
        pl.BlockSpec((D, GRP), lambda k: (0, k)),
        pl.BlockSpec((D, GRP), lambda k: (0, k)),
    ],
    out_specs=pl.BlockSpec((PAIR, 2 * D), lambda k: (k, 0)),
    out_shape=jax.ShapeDtypeStruct((PROWS, 2 * D), jnp.int32),
)


def _rotate_body(sup_ref, r_ref, col_ref, ent_pk, cs_t,
                 out, idx_e, idx_r, col_v, pk, cs, sem):
    wid = lax.axis_index("s") * NC + lax.axis_index("c")
    row0 = wid * CPW
    pltpu.sync_copy(sup_ref.at[pl.ds(row0, CPW)], idx_e)
    pltpu.sync_copy(r_ref.at[pl.ds(row0, CPW)], idx_r)
    pltpu.sync_copy(col_ref.at[pl.ds(row0, CPW)], col_v)
    hi_mask = jnp.full((L,), -65536, jnp.int32)
    for j in range(CPW):
        cps = [
            pltpu.async_copy(ent_pk.at[idx_e.at[j]], pk, sem),
            pltpu.async_copy(cs_t.at[idx_r.at[j]], cs, sem),
        ]
        for c in cps:
            c.wait()

        def body(i, carry):
            zl = jnp.zeros((L,), jnp.int32)
            p = plsc.load_gather(col_v, [zl + j, zl + i])
            m = p > 0
            for k in range(D // L):
                sl = pl.ds(k * L, L)
                sh = pl.ds(D + k * L, L)
                v = jnp.where(m, pk[i, sh], pk[i, sl])
                a = plsc.bitcast(v << 16, jnp.float32)
                b = plsc.bitcast(v & hi_mask, jnp.float32)
                c = cs[i, sl]
                s = cs[i, sh]
                cs[i, sl] = a * c - b * s
                cs[i, sh] = a * s + b * c
            return carry

        lax.fori_loop(0, CHUNK, body, 0)
        base = (row0 + j) * CHUNK
        pltpu.sync_copy(cs, out.at[pl.ds(base, CHUNK)])


_rotate = functools.partial(
    pl.kernel,
    out_type=jax.ShapeDtypeStruct((BATCH, 2 * D), jnp.float32),
    mesh=plsc.VectorSubcoreMesh(
        core_axis_name="c", subcore_axis_name="s", num_cores=NC, num_subcores=NS),
    scratch_types=[
        pltpu.VMEM((CPW, CHUNK), jnp.int32),
        pltpu.VMEM((CPW, CHUNK), jnp.int32),
        pltpu.VMEM((CPW, CHUNK), jnp.int32),
        pltpu.VMEM((CHUNK, 2 * D), jnp.int32),
        pltpu.VMEM((CHUNK, 2 * D), jnp.float32),
        pltpu.SemaphoreType.DMA,
    ],
    compiler_params=pltpu.CompilerParams(needs_layout_passes=False),
)(_rotate_body)


def kernel(e1, r, entity_real, entity_img, relation):
    e1 = e1.astype(jnp.int32)
    r = r.astype(jnp.int32).reshape(N_CHUNKS, CHUNK)
    off = e1 % GRP
    sup = (PAIR * (e1 // GRP) + (off % PAIR)).reshape(N_CHUNKS, CHUNK)
    col = jnp.where(off >= PAIR, D, 0).astype(jnp.int32).reshape(N_CHUNKS, CHUNK)
    packed = _pack(entity_real.T, entity_img.T)
    cs_t = _trig(relation)
    out = _rotate(sup, r, col, packed, cs_t)
    return out[:, :D], out[:, D:]
